# Initial kernel scaffold; baseline (speedup 1.0000x reference)
import jax
import jax.numpy as jnp
from jax.experimental import pallas as pl


def kernel(atom_pos, node_attrs, edge_index, shifts, W_embed, W_up, Wr1, Wr2, W_msg, W_skip):
    raise NotImplementedError("write your pallas kernel here")


# speedup vs baseline: 27.7510x; 27.7510x over previous
import functools

import jax
import jax.numpy as jnp
import numpy as np
from jax import lax
from jax.experimental import pallas as pl
from jax.experimental.pallas import tpu as pltpu
from jax.experimental.pallas import tpu_sc as plsc

N = 10000
E = 320000
C = 64
SH = 4
W128 = 128
R_CUT = 5.0

NC = 2
NS = 16
NW = NC * NS
EPW = E // NW
GCH = 80
NCH = EPW // GCH

_SC_MESH = plsc.VectorSubcoreMesh(core_axis_name="c", subcore_axis_name="s")


@functools.partial(
    pl.kernel,
    mesh=_SC_MESH,
    out_type=jax.ShapeDtypeStruct((E, W128), jnp.float32),
    scratch_types=[
        pltpu.VMEM((GCH,), jnp.int32),
        pltpu.VMEM((GCH, W128), jnp.float32),
        pltpu.VMEM_SHARED((N, W128), jnp.float32),
        pltpu.SemaphoreType.DMA,
    ],
)
def _sc_gather(tab_hbm, idx_hbm, out_hbm, si_v, rows_v, tab_sh, sem):
    sid = lax.axis_index("s")
    wid = sid * NC + lax.axis_index("c")
    base = wid * EPW

    @pl.when(sid == 0)
    def _():
        pltpu.sync_copy(tab_hbm, tab_sh)

    plsc.subcore_barrier()

    def body(j, carry):
        off = base + j * GCH
        pltpu.sync_copy(idx_hbm.at[pl.ds(off, GCH)], si_v)
        pltpu.async_copy(tab_sh.at[si_v], rows_v, sem).wait()
        pltpu.sync_copy(rows_v, out_hbm.at[pl.ds(off, GCH)])
        return carry

    lax.fori_loop(0, NCH, body, 0)


@functools.partial(
    pl.kernel,
    mesh=_SC_MESH,
    out_type=jax.ShapeDtypeStruct((NC, N, W128), jnp.float32),
    scratch_types=[
        pltpu.VMEM((NCH, GCH), jnp.int32),
        pltpu.VMEM((GCH, W128), jnp.float32),
        pltpu.VMEM_SHARED((N, W128), jnp.float32),
        pltpu.SemaphoreType.DMA,
    ],
)
def _sc_scatter(msg_hbm, dst_hbm, zeros_hbm, out_hbm, idx_v, pay_v, acc_sh, sem):
    cid = lax.axis_index("c")
    sid = lax.axis_index("s")
    wid = sid * NC + cid
    base = wid * EPW

    @pl.when(sid == 0)
    def _():
        pltpu.sync_copy(zeros_hbm, acc_sh)

    plsc.subcore_barrier()

    def body(j, carry):
        pltpu.sync_copy(dst_hbm.at[pl.ds(base + j * GCH, GCH)], idx_v.at[j])
        pltpu.sync_copy(msg_hbm.at[pl.ds(base + j * GCH, GCH)], pay_v)
        pltpu.sync_copy(pay_v, acc_sh.at[idx_v.at[j]], add=True)
        return carry

    lax.fori_loop(0, NCH, body, 0)

    plsc.subcore_barrier()

    @pl.when(sid == 0)
    def _():
        pltpu.sync_copy(acc_sh, out_hbm.at[cid])


BE_G = 4000
BE_E = 2000
BN = 2000


def _tc_geom(ps, pd, sh8):
    def body(ps_ref, pd_ref, sh_ref, out_ref):
        vec = pd_ref[:, 0:8] - ps_ref[:, 0:8] + sh_ref[...]
        r2 = jnp.sum(vec * vec, axis=1, keepdims=True) + 1e-12
        r = jnp.sqrt(r2)
        unit = vec / r
        nvec = lax.broadcasted_iota(jnp.int32, (1, 8), 1).astype(jnp.float32) + 1.0
        bess = np.float32(np.sqrt(2.0 / R_CUT)) * jnp.sin(nvec * np.float32(np.pi / R_CUT) * r) / r
        u = r * np.float32(1.0 / R_CUT)
        u2 = u * u
        u4 = u2 * u2
        u5 = u4 * u
        u6 = u5 * u
        u7 = u6 * u
        env = (1.0 - 21.0 * u5 + 35.0 * u6 - 15.0 * u7) * (u < 1.0).astype(jnp.float32)
        ef = bess * env
        ones = jnp.ones_like(r)
        ea = jnp.concatenate([ones, np.float32(np.sqrt(3.0)) * unit[:, 0:3]], axis=1)
        out_ref[...] = jnp.concatenate([ea, ef, jnp.zeros_like(ea)], axis=1)

    return pl.pallas_call(
        body,
        grid=(E // BE_G,),
        in_specs=[pl.BlockSpec((BE_G, W128), lambda i: (i, 0)),
                  pl.BlockSpec((BE_G, W128), lambda i: (i, 0)),
                  pl.BlockSpec((BE_G, 8), lambda i: (i, 0))],
        out_specs=pl.BlockSpec((BE_G, 16), lambda i: (i, 0)),
        out_shape=jax.ShapeDtypeStruct((E, 16), jnp.float32),
    )(ps, pd, sh8)


def _tc_node_init(na, we, wu0):
    def body(na_ref, we_ref, wu_ref, nf_ref, tab_ref):
        nf = jnp.dot(na_ref[...], we_ref[...], preferred_element_type=jnp.float32)
        nf_ref[...] = nf
        h = jnp.dot(nf, wu_ref[...], preferred_element_type=jnp.float32)
        tab_ref[...] = jnp.concatenate([h, jnp.zeros_like(h)], axis=1)

    return pl.pallas_call(
        body,
        grid=(N // BN,),
        in_specs=[pl.BlockSpec((BN, 10), lambda i: (i, 0)),
                  pl.BlockSpec((10, C), lambda i: (0, 0)),
                  pl.BlockSpec((C, C), lambda i: (0, 0))],
        out_specs=(pl.BlockSpec((BN, C), lambda i: (i, 0)),
                   pl.BlockSpec((BN, W128), lambda i: (i, 0))),
        out_shape=(jax.ShapeDtypeStruct((N, C), jnp.float32),
                   jax.ShapeDtypeStruct((N, W128), jnp.float32)),
    )(na, we, wu0)


def _tc_edge(g, eaef, wr1, wr2, wcat):
    def body(g_ref, ee_ref, wr1_ref, wr2_ref, wc_ref, out_ref):
        ee = ee_ref[...]
        ef = ee[:, 4:12]
        hid = jnp.dot(ef, wr1_ref[...], preferred_element_type=jnp.float32)
        hid = hid * jax.nn.sigmoid(hid)
        rad = jnp.dot(hid, wr2_ref[...], preferred_element_type=jnp.float32)
        hm = g_ref[:, 0:C] * rad
        m4 = jnp.dot(hm, wc_ref[...], preferred_element_type=jnp.float32)
        msg = (ee[:, 0:1] * m4[:, 0:C]
               + ee[:, 1:2] * m4[:, C:2 * C]
               + ee[:, 2:3] * m4[:, 2 * C:3 * C]
               + ee[:, 3:4] * m4[:, 3 * C:4 * C])
        out_ref[...] = jnp.concatenate([msg, jnp.zeros_like(msg)], axis=1)

    return pl.pallas_call(
        body,
        grid=(E // BE_E,),
        in_specs=[pl.BlockSpec((BE_E, W128), lambda i: (i, 0)),
                  pl.BlockSpec((BE_E, 16), lambda i: (i, 0)),
                  pl.BlockSpec((8, 16), lambda i: (0, 0)),
                  pl.BlockSpec((16, C), lambda i: (0, 0)),
                  pl.BlockSpec((C, SH * C), lambda i: (0, 0))],
        out_specs=pl.BlockSpec((BE_E, W128), lambda i: (i, 0)),
        out_shape=jax.ShapeDtypeStruct((E, W128), jnp.float32),
    )(g, eaef, wr1, wr2, wcat)


def _tc_combine(nf, parts, wskip, wup_next):
    def body(nf_ref, p0_ref, p1_ref, ws_ref, wu_ref, nfn_ref, tab_ref):
        agg = p0_ref[0][:, 0:C] + p1_ref[0][:, 0:C]
        nfn = agg + jnp.dot(nf_ref[...], ws_ref[...], preferred_element_type=jnp.float32)
        nfn_ref[...] = nfn
        h = jnp.dot(nfn, wu_ref[...], preferred_element_type=jnp.float32)
        tab_ref[...] = jnp.concatenate([h, jnp.zeros_like(h)], axis=1)

    return pl.pallas_call(
        body,
        grid=(N // BN,),
        in_specs=[pl.BlockSpec((BN, C), lambda i: (i, 0)),
                  pl.BlockSpec((1, BN, W128), lambda i: (0, i, 0)),
                  pl.BlockSpec((1, BN, W128), lambda i: (1, i, 0)),
                  pl.BlockSpec((C, C), lambda i: (0, 0)),
                  pl.BlockSpec((C, C), lambda i: (0, 0))],
        out_specs=(pl.BlockSpec((BN, C), lambda i: (i, 0)),
                   pl.BlockSpec((BN, W128), lambda i: (i, 0))),
        out_shape=(jax.ShapeDtypeStruct((N, C), jnp.float32),
                   jax.ShapeDtypeStruct((N, W128), jnp.float32)),
    )(nf, parts, parts, wskip, wup_next)


def kernel(atom_pos, node_attrs, edge_index, shifts, W_embed, W_up, Wr1, Wr2, W_msg, W_skip):
    src = edge_index[0].astype(jnp.int32)
    dst = edge_index[1].astype(jnp.int32)
    pos128 = jnp.pad(atom_pos, ((0, 0), (0, W128 - 3)))
    sh8 = jnp.pad(shifts, ((0, 0), (0, 5)))
    zeros_nw = jnp.zeros((N, W128), jnp.float32)

    ps = _sc_gather(pos128, src)
    pd = _sc_gather(pos128, dst)
    eaef = _tc_geom(ps, pd, sh8)
    nf, tab = _tc_node_init(node_attrs, W_embed, W_up[0])

    outs = []
    for i in range(2):
        g = _sc_gather(tab, src)
        msg = _tc_edge(g, eaef, Wr1[i], Wr2[i], W_msg[i].reshape(C, SH * C))
        parts = _sc_scatter(msg, dst, zeros_nw)
        nf, tab = _tc_combine(nf, parts, W_skip[i], W_up[1])
        outs.append(nf)
    return jnp.concatenate(outs, axis=-1)


# speedup vs baseline: 30.6271x; 1.1036x over previous
import functools

import jax
import jax.numpy as jnp
import numpy as np
from jax import lax
from jax.experimental import pallas as pl
from jax.experimental.pallas import tpu as pltpu
from jax.experimental.pallas import tpu_sc as plsc

N = 10000
E = 320000
C = 64
SH = 4
W128 = 128
R_CUT = 5.0

NC = 2
NS = 16
NW = NC * NS
EPW = E // NW
GCH = 80
NCH = EPW // GCH
GPC = 25
NGRP = NCH // GPC
NBUF = 6
LOOK = 3

_SC_MESH = plsc.VectorSubcoreMesh(core_axis_name="c", subcore_axis_name="s")


@functools.partial(
    pl.kernel,
    mesh=_SC_MESH,
    out_type=jax.ShapeDtypeStruct((E, W128), jnp.float32),
    scratch_types=(
        [pltpu.VMEM((EPW,), jnp.int32)]
        + [pltpu.VMEM((GCH, W128), jnp.float32) for _ in range(NBUF)]
        + [pltpu.SemaphoreType.DMA for _ in range(2 * NBUF)]
    ),
)
def _sc_gather(tab_hbm, idx_hbm, out_hbm, idx_v, *rest):
    rows = rest[:NBUF]
    gsems = rest[NBUF:2 * NBUF]
    wsems = rest[2 * NBUF:]
    sid = lax.axis_index("s")
    wid = sid * NC + lax.axis_index("c")
    base = wid * EPW

    pltpu.sync_copy(idx_hbm.at[pl.ds(base, EPW)], idx_v)

    def group(g, carry):
        gh = [None] * GPC
        wh = [None] * GPC
        for t in range(GPC + LOOK):
            if t < GPC:
                b = t % NBUF
                if t >= NBUF:
                    wh[t - NBUF].wait()
                j = g * GPC + t
                gh[t] = pltpu.async_copy(
                    tab_hbm.at[idx_v.at[pl.ds(j * GCH, GCH)]], rows[b], gsems[b])
            td = t - LOOK
            if 0 <= td < GPC:
                gh[td].wait()
                jd = g * GPC + td
                wh[td] = pltpu.async_copy(
                    rows[td % NBUF], out_hbm.at[pl.ds(base + jd * GCH, GCH)],
                    wsems[td % NBUF])
        for td in range(GPC - NBUF, GPC):
            wh[td].wait()
        return carry

    lax.fori_loop(0, NGRP, group, 0)


NBUF2 = 3
LOOK2 = 2
SNBUF = 3
SLOOK = 2


@functools.partial(
    pl.kernel,
    mesh=_SC_MESH,
    out_type=(jax.ShapeDtypeStruct((E, W128), jnp.float32),
              jax.ShapeDtypeStruct((E, W128), jnp.float32)),
    scratch_types=(
        [pltpu.VMEM((EPW,), jnp.int32), pltpu.VMEM((EPW,), jnp.int32)]
        + [pltpu.VMEM((GCH, W128), jnp.float32) for _ in range(2 * NBUF2)]
        + [pltpu.SemaphoreType.DMA for _ in range(4 * NBUF2)]
    ),
)
def _sc_gather_pair(tab_hbm, idxa_hbm, idxb_hbm, outa_hbm, outb_hbm,
                    idxa_v, idxb_v, *rest):
    rows_a = rest[:NBUF2]
    rows_b = rest[NBUF2:2 * NBUF2]
    sems = rest[2 * NBUF2:]
    gsems_a = sems[:NBUF2]
    gsems_b = sems[NBUF2:2 * NBUF2]
    wsems_a = sems[2 * NBUF2:3 * NBUF2]
    wsems_b = sems[3 * NBUF2:]
    sid = lax.axis_index("s")
    wid = sid * NC + lax.axis_index("c")
    base = wid * EPW

    pltpu.sync_copy(idxa_hbm.at[pl.ds(base, EPW)], idxa_v)
    pltpu.sync_copy(idxb_hbm.at[pl.ds(base, EPW)], idxb_v)

    def group(g, carry):
        gh = [None] * GPC
        wh = [None] * GPC
        for t in range(GPC + LOOK2):
            if t < GPC:
                b = t % NBUF2
                if t >= NBUF2:
                    for h in wh[t - NBUF2]:
                        h.wait()
                j = g * GPC + t
                sl = pl.ds(j * GCH, GCH)
                gh[t] = (
                    pltpu.async_copy(tab_hbm.at[idxa_v.at[sl]], rows_a[b], gsems_a[b]),
                    pltpu.async_copy(tab_hbm.at[idxb_v.at[sl]], rows_b[b], gsems_b[b]),
                )
            td = t - LOOK2
            if 0 <= td < GPC:
                for h in gh[td]:
                    h.wait()
                jd = g * GPC + td
                osl = pl.ds(base + jd * GCH, GCH)
                bd = td % NBUF2
                wh[td] = (
                    pltpu.async_copy(rows_a[bd], outa_hbm.at[osl], wsems_a[bd]),
                    pltpu.async_copy(rows_b[bd], outb_hbm.at[osl], wsems_b[bd]),
                )
        for td in range(GPC - NBUF2, GPC):
            for h in wh[td]:
                h.wait()
        return carry

    lax.fori_loop(0, NGRP, group, 0)


@functools.partial(
    pl.kernel,
    mesh=_SC_MESH,
    out_type=jax.ShapeDtypeStruct((NC, N, W128), jnp.float32),
    scratch_types=(
        [pltpu.VMEM((NCH, GCH), jnp.int32)]
        + [pltpu.VMEM((GCH, W128), jnp.float32) for _ in range(SNBUF)]
        + [pltpu.VMEM_SHARED((N, W128), jnp.float32)]
        + [pltpu.SemaphoreType.DMA for _ in range(3 * SNBUF)]
    ),
)
def _sc_scatter(msg_hbm, dst_hbm, zeros_hbm, out_hbm, idx_v, *rest):
    pay = rest[:SNBUF]
    acc_sh = rest[SNBUF]
    isems = rest[SNBUF + 1:SNBUF + 1 + SNBUF]
    lsems = rest[SNBUF + 1 + SNBUF:SNBUF + 1 + 2 * SNBUF]
    ssems = rest[SNBUF + 1 + 2 * SNBUF:]
    cid = lax.axis_index("c")
    sid = lax.axis_index("s")
    wid = sid * NC + cid
    base = wid * EPW

    @pl.when(sid == 0)
    def _():
        pltpu.sync_copy(zeros_hbm, acc_sh)

    plsc.subcore_barrier()

    def group(g, carry):
        ih = [None] * GPC
        lh = [None] * GPC
        sh = [None] * GPC
        for t in range(GPC + SLOOK):
            if t < GPC:
                b = t % SNBUF
                if t >= SNBUF:
                    sh[t - SNBUF].wait()
                j = g * GPC + t
                ih[t] = pltpu.async_copy(
                    dst_hbm.at[pl.ds(base + j * GCH, GCH)], idx_v.at[j], isems[b])
                lh[t] = pltpu.async_copy(
                    msg_hbm.at[pl.ds(base + j * GCH, GCH)], pay[b], lsems[b])
            td = t - SLOOK
            if 0 <= td < GPC:
                ih[td].wait()
                lh[td].wait()
                jd = g * GPC + td
                sh[td] = pltpu.async_copy(
                    pay[td % SNBUF], acc_sh.at[idx_v.at[jd]], ssems[td % SNBUF],
                    add=True)
        for td in range(GPC - SNBUF, GPC):
            sh[td].wait()
        return carry

    lax.fori_loop(0, NGRP, group, 0)

    plsc.subcore_barrier()

    @pl.when(sid == 0)
    def _():
        pltpu.sync_copy(acc_sh, out_hbm.at[cid])


BE_G = 4000
BE_E = 2000
BN = 2000


def _tc_geom(ps, pd, sh8):
    def body(ps_ref, pd_ref, sh_ref, out_ref):
        vec = pd_ref[:, 0:8] - ps_ref[:, 0:8] + sh_ref[...]
        r2 = jnp.sum(vec * vec, axis=1, keepdims=True) + 1e-12
        r = jnp.sqrt(r2)
        unit = vec / r
        nvec = lax.broadcasted_iota(jnp.int32, (1, 8), 1).astype(jnp.float32) + 1.0
        bess = np.float32(np.sqrt(2.0 / R_CUT)) * jnp.sin(nvec * np.float32(np.pi / R_CUT) * r) / r
        u = r * np.float32(1.0 / R_CUT)
        u2 = u * u
        u4 = u2 * u2
        u5 = u4 * u
        u6 = u5 * u
        u7 = u6 * u
        env = (1.0 - 21.0 * u5 + 35.0 * u6 - 15.0 * u7) * (u < 1.0).astype(jnp.float32)
        ef = bess * env
        ones = jnp.ones_like(r)
        ea = jnp.concatenate([ones, np.float32(np.sqrt(3.0)) * unit[:, 0:3]], axis=1)
        out_ref[...] = jnp.concatenate([ea, ef, jnp.zeros_like(ea)], axis=1)

    return pl.pallas_call(
        body,
        grid=(E // BE_G,),
        in_specs=[pl.BlockSpec((BE_G, W128), lambda i: (i, 0)),
                  pl.BlockSpec((BE_G, W128), lambda i: (i, 0)),
                  pl.BlockSpec((BE_G, 8), lambda i: (i, 0))],
        out_specs=pl.BlockSpec((BE_G, 16), lambda i: (i, 0)),
        out_shape=jax.ShapeDtypeStruct((E, 16), jnp.float32),
    )(ps, pd, sh8)


def _tc_node_init(na, we, wu0):
    def body(na_ref, we_ref, wu_ref, nf_ref, tab_ref):
        nf = jnp.dot(na_ref[...], we_ref[...], preferred_element_type=jnp.float32)
        nf_ref[...] = nf
        h = jnp.dot(nf, wu_ref[...], preferred_element_type=jnp.float32)
        tab_ref[...] = jnp.concatenate([h, jnp.zeros_like(h)], axis=1)

    return pl.pallas_call(
        body,
        grid=(N // BN,),
        in_specs=[pl.BlockSpec((BN, 10), lambda i: (i, 0)),
                  pl.BlockSpec((10, C), lambda i: (0, 0)),
                  pl.BlockSpec((C, C), lambda i: (0, 0))],
        out_specs=(pl.BlockSpec((BN, C), lambda i: (i, 0)),
                   pl.BlockSpec((BN, W128), lambda i: (i, 0))),
        out_shape=(jax.ShapeDtypeStruct((N, C), jnp.float32),
                   jax.ShapeDtypeStruct((N, W128), jnp.float32)),
    )(na, we, wu0)


def _tc_edge(g, eaef, wr1, wr2, wcat):
    def body(g_ref, ee_ref, wr1_ref, wr2_ref, wc_ref, out_ref):
        ee = ee_ref[...]
        ef = ee[:, 4:12]
        hid = jnp.dot(ef, wr1_ref[...], preferred_element_type=jnp.float32)
        hid = hid * jax.nn.sigmoid(hid)
        rad = jnp.dot(hid, wr2_ref[...], preferred_element_type=jnp.float32)
        hm = g_ref[:, 0:C] * rad
        m4 = jnp.dot(hm, wc_ref[...], preferred_element_type=jnp.float32)
        msg = (ee[:, 0:1] * m4[:, 0:C]
               + ee[:, 1:2] * m4[:, C:2 * C]
               + ee[:, 2:3] * m4[:, 2 * C:3 * C]
               + ee[:, 3:4] * m4[:, 3 * C:4 * C])
        out_ref[...] = jnp.concatenate([msg, jnp.zeros_like(msg)], axis=1)

    return pl.pallas_call(
        body,
        grid=(E // BE_E,),
        in_specs=[pl.BlockSpec((BE_E, W128), lambda i: (i, 0)),
                  pl.BlockSpec((BE_E, 16), lambda i: (i, 0)),
                  pl.BlockSpec((8, 16), lambda i: (0, 0)),
                  pl.BlockSpec((16, C), lambda i: (0, 0)),
                  pl.BlockSpec((C, SH * C), lambda i: (0, 0))],
        out_specs=pl.BlockSpec((BE_E, W128), lambda i: (i, 0)),
        out_shape=jax.ShapeDtypeStruct((E, W128), jnp.float32),
    )(g, eaef, wr1, wr2, wcat)


def _tc_combine(nf, parts, wskip, wup_next):
    def body(nf_ref, p0_ref, p1_ref, ws_ref, wu_ref, nfn_ref, tab_ref):
        agg = p0_ref[0][:, 0:C] + p1_ref[0][:, 0:C]
        nfn = agg + jnp.dot(nf_ref[...], ws_ref[...], preferred_element_type=jnp.float32)
        nfn_ref[...] = nfn
        h = jnp.dot(nfn, wu_ref[...], preferred_element_type=jnp.float32)
        tab_ref[...] = jnp.concatenate([h, jnp.zeros_like(h)], axis=1)

    return pl.pallas_call(
        body,
        grid=(N // BN,),
        in_specs=[pl.BlockSpec((BN, C), lambda i: (i, 0)),
                  pl.BlockSpec((1, BN, W128), lambda i: (0, i, 0)),
                  pl.BlockSpec((1, BN, W128), lambda i: (1, i, 0)),
                  pl.BlockSpec((C, C), lambda i: (0, 0)),
                  pl.BlockSpec((C, C), lambda i: (0, 0))],
        out_specs=(pl.BlockSpec((BN, C), lambda i: (i, 0)),
                   pl.BlockSpec((BN, W128), lambda i: (i, 0))),
        out_shape=(jax.ShapeDtypeStruct((N, C), jnp.float32),
                   jax.ShapeDtypeStruct((N, W128), jnp.float32)),
    )(nf, parts, parts, wskip, wup_next)


def kernel(atom_pos, node_attrs, edge_index, shifts, W_embed, W_up, Wr1, Wr2, W_msg, W_skip):
    src = edge_index[0].astype(jnp.int32)
    dst = edge_index[1].astype(jnp.int32)
    pos128 = jnp.pad(atom_pos, ((0, 0), (0, W128 - 3)))
    sh8 = jnp.pad(shifts, ((0, 0), (0, 5)))
    zeros_nw = jnp.zeros((N, W128), jnp.float32)

    ps, pd = _sc_gather_pair(pos128, src, dst)
    eaef = _tc_geom(ps, pd, sh8)
    nf, tab = _tc_node_init(node_attrs, W_embed, W_up[0])

    wcats = W_msg.reshape(-1, C, SH * C)
    wup1 = W_up[1]

    def layer_step(carry, xs):
        nf_c, tab_c = carry
        wr1, wr2, wcat, wskip = xs
        g = _sc_gather(tab_c, src)
        msg = _tc_edge(g, eaef, wr1, wr2, wcat)
        parts = _sc_scatter(msg, dst, zeros_nw)
        nf_n, tab_n = _tc_combine(nf_c, parts, wskip, wup1)
        return (nf_n, tab_n), nf_n

    _, ys = lax.scan(layer_step, (nf, tab), (Wr1, Wr2, wcats, W_skip))
    return jnp.concatenate([ys[0], ys[1]], axis=-1)


# speedup vs baseline: 32.3441x; 1.0561x over previous
import functools

import jax
import jax.numpy as jnp
import numpy as np
from jax import lax
from jax.experimental import pallas as pl
from jax.experimental.pallas import tpu as pltpu
from jax.experimental.pallas import tpu_sc as plsc

N = 10000
E = 320000
C = 64
SH = 4
W128 = 128
R_CUT = 5.0

NC = 2
NS = 16
NW = NC * NS
EPW = E // NW
GCH = 80
NCH = EPW // GCH
GPC = 25
NGRP = NCH // GPC
NBUF = 6
LOOK = 3

_SC_MESH = plsc.VectorSubcoreMesh(core_axis_name="c", subcore_axis_name="s")


@functools.partial(
    pl.kernel,
    mesh=_SC_MESH,
    out_type=jax.ShapeDtypeStruct((E, W128), jnp.float32),
    scratch_types=(
        [pltpu.VMEM((EPW,), jnp.int32)]
        + [pltpu.VMEM((GCH, W128), jnp.float32) for _ in range(NBUF)]
        + [pltpu.SemaphoreType.DMA for _ in range(2 * NBUF)]
    ),
)
def _sc_gather(tab_hbm, idx_hbm, out_hbm, idx_v, *rest):
    rows = rest[:NBUF]
    gsems = rest[NBUF:2 * NBUF]
    wsems = rest[2 * NBUF:]
    sid = lax.axis_index("s")
    wid = sid * NC + lax.axis_index("c")
    base = wid * EPW

    pltpu.sync_copy(idx_hbm.at[pl.ds(base, EPW)], idx_v)

    def group(g, carry):
        gh = [None] * GPC
        wh = [None] * GPC
        for t in range(GPC + LOOK):
            if t < GPC:
                b = t % NBUF
                if t >= NBUF:
                    wh[t - NBUF].wait()
                j = g * GPC + t
                gh[t] = pltpu.async_copy(
                    tab_hbm.at[idx_v.at[pl.ds(j * GCH, GCH)]], rows[b], gsems[b])
            td = t - LOOK
            if 0 <= td < GPC:
                gh[td].wait()
                jd = g * GPC + td
                wh[td] = pltpu.async_copy(
                    rows[td % NBUF], out_hbm.at[pl.ds(base + jd * GCH, GCH)],
                    wsems[td % NBUF])
        for td in range(GPC - NBUF, GPC):
            wh[td].wait()
        return carry

    lax.fori_loop(0, NGRP, group, 0)


NBUF2 = 3
LOOK2 = 2
SNBUF = 3
SLOOK = 2


@functools.partial(
    pl.kernel,
    mesh=_SC_MESH,
    out_type=(jax.ShapeDtypeStruct((E, W128), jnp.float32),
              jax.ShapeDtypeStruct((E, W128), jnp.float32)),
    scratch_types=(
        [pltpu.VMEM((EPW,), jnp.int32), pltpu.VMEM((EPW,), jnp.int32)]
        + [pltpu.VMEM((GCH, W128), jnp.float32) for _ in range(2 * NBUF2)]
        + [pltpu.SemaphoreType.DMA for _ in range(4 * NBUF2)]
    ),
)
def _sc_gather_pair(tab_hbm, idxa_hbm, idxb_hbm, outa_hbm, outb_hbm,
                    idxa_v, idxb_v, *rest):
    rows_a = rest[:NBUF2]
    rows_b = rest[NBUF2:2 * NBUF2]
    sems = rest[2 * NBUF2:]
    gsems_a = sems[:NBUF2]
    gsems_b = sems[NBUF2:2 * NBUF2]
    wsems_a = sems[2 * NBUF2:3 * NBUF2]
    wsems_b = sems[3 * NBUF2:]
    sid = lax.axis_index("s")
    wid = sid * NC + lax.axis_index("c")
    base = wid * EPW

    pltpu.sync_copy(idxa_hbm.at[pl.ds(base, EPW)], idxa_v)
    pltpu.sync_copy(idxb_hbm.at[pl.ds(base, EPW)], idxb_v)

    def group(g, carry):
        gh = [None] * GPC
        wh = [None] * GPC
        for t in range(GPC + LOOK2):
            if t < GPC:
                b = t % NBUF2
                if t >= NBUF2:
                    for h in wh[t - NBUF2]:
                        h.wait()
                j = g * GPC + t
                sl = pl.ds(j * GCH, GCH)
                gh[t] = (
                    pltpu.async_copy(tab_hbm.at[idxa_v.at[sl]], rows_a[b], gsems_a[b]),
                    pltpu.async_copy(tab_hbm.at[idxb_v.at[sl]], rows_b[b], gsems_b[b]),
                )
            td = t - LOOK2
            if 0 <= td < GPC:
                for h in gh[td]:
                    h.wait()
                jd = g * GPC + td
                osl = pl.ds(base + jd * GCH, GCH)
                bd = td % NBUF2
                wh[td] = (
                    pltpu.async_copy(rows_a[bd], outa_hbm.at[osl], wsems_a[bd]),
                    pltpu.async_copy(rows_b[bd], outb_hbm.at[osl], wsems_b[bd]),
                )
        for td in range(GPC - NBUF2, GPC):
            for h in wh[td]:
                h.wait()
        return carry

    lax.fori_loop(0, NGRP, group, 0)


@functools.partial(
    pl.kernel,
    mesh=_SC_MESH,
    out_type=jax.ShapeDtypeStruct((NC, N, W128), jnp.float32),
    scratch_types=(
        [pltpu.VMEM((NCH, GCH), jnp.int32)]
        + [pltpu.VMEM((GCH, W128), jnp.float32) for _ in range(SNBUF)]
        + [pltpu.VMEM_SHARED((N, W128), jnp.float32)]
        + [pltpu.SemaphoreType.DMA for _ in range(3 * SNBUF)]
    ),
)
def _sc_scatter(msg_hbm, dst_hbm, zeros_hbm, out_hbm, idx_v, *rest):
    pay = rest[:SNBUF]
    acc_sh = rest[SNBUF]
    isems = rest[SNBUF + 1:SNBUF + 1 + SNBUF]
    lsems = rest[SNBUF + 1 + SNBUF:SNBUF + 1 + 2 * SNBUF]
    ssems = rest[SNBUF + 1 + 2 * SNBUF:]
    cid = lax.axis_index("c")
    sid = lax.axis_index("s")
    wid = sid * NC + cid
    base = wid * EPW

    @pl.when(sid == 0)
    def _():
        pltpu.sync_copy(zeros_hbm, acc_sh)

    plsc.subcore_barrier()

    def group(g, carry):
        ih = [None] * GPC
        lh = [None] * GPC
        sh = [None] * GPC
        for t in range(GPC + SLOOK):
            if t < GPC:
                b = t % SNBUF
                if t >= SNBUF:
                    sh[t - SNBUF].wait()
                j = g * GPC + t
                ih[t] = pltpu.async_copy(
                    dst_hbm.at[pl.ds(base + j * GCH, GCH)], idx_v.at[j], isems[b])
                lh[t] = pltpu.async_copy(
                    msg_hbm.at[pl.ds(base + j * GCH, GCH)], pay[b], lsems[b])
            td = t - SLOOK
            if 0 <= td < GPC:
                ih[td].wait()
                lh[td].wait()
                jd = g * GPC + td
                sh[td] = pltpu.async_copy(
                    pay[td % SNBUF], acc_sh.at[idx_v.at[jd]], ssems[td % SNBUF],
                    add=True)
        for td in range(GPC - SNBUF, GPC):
            sh[td].wait()
        return carry

    lax.fori_loop(0, NGRP, group, 0)

    plsc.subcore_barrier()

    @pl.when(sid == 0)
    def _():
        pltpu.sync_copy(acc_sh, out_hbm.at[cid])


BE_G = 4000
BE_E = 4000
BN = 2000


def _tc_geom(ps, pd, sh8):
    def body(ps_ref, pd_ref, sh_ref, out_ref):
        vec = pd_ref[:, 0:8] - ps_ref[:, 0:8] + sh_ref[...]
        r2 = jnp.sum(vec * vec, axis=1, keepdims=True) + 1e-12
        r = jnp.sqrt(r2)
        unit = vec / r
        nvec = lax.broadcasted_iota(jnp.int32, (1, 8), 1).astype(jnp.float32) + 1.0
        bess = np.float32(np.sqrt(2.0 / R_CUT)) * jnp.sin(nvec * np.float32(np.pi / R_CUT) * r) / r
        u = r * np.float32(1.0 / R_CUT)
        u2 = u * u
        u4 = u2 * u2
        u5 = u4 * u
        u6 = u5 * u
        u7 = u6 * u
        env = (1.0 - 21.0 * u5 + 35.0 * u6 - 15.0 * u7) * (u < 1.0).astype(jnp.float32)
        ef = bess * env
        ones = jnp.ones_like(r)
        ea = jnp.concatenate([ones, np.float32(np.sqrt(3.0)) * unit[:, 0:3]], axis=1)
        out_ref[...] = jnp.concatenate([ea, ef, jnp.zeros_like(ea)], axis=1)

    return pl.pallas_call(
        body,
        grid=(E // BE_G,),
        in_specs=[pl.BlockSpec((BE_G, W128), lambda i: (i, 0)),
                  pl.BlockSpec((BE_G, W128), lambda i: (i, 0)),
                  pl.BlockSpec((BE_G, 8), lambda i: (i, 0))],
        out_specs=pl.BlockSpec((BE_G, 16), lambda i: (i, 0)),
        out_shape=jax.ShapeDtypeStruct((E, 16), jnp.float32),
    )(ps, pd, sh8)


def _tc_node_init(na, we, wu0):
    def body(na_ref, we_ref, wu_ref, nf_ref, tab_ref):
        nf = jnp.dot(na_ref[...], we_ref[...], preferred_element_type=jnp.float32)
        nf_ref[...] = nf
        h = jnp.dot(nf, wu_ref[...], preferred_element_type=jnp.float32)
        tab_ref[...] = jnp.concatenate([h, jnp.zeros_like(h)], axis=1)

    return pl.pallas_call(
        body,
        grid=(N // BN,),
        in_specs=[pl.BlockSpec((BN, 10), lambda i: (i, 0)),
                  pl.BlockSpec((10, C), lambda i: (0, 0)),
                  pl.BlockSpec((C, C), lambda i: (0, 0))],
        out_specs=(pl.BlockSpec((BN, C), lambda i: (i, 0)),
                   pl.BlockSpec((BN, W128), lambda i: (i, 0))),
        out_shape=(jax.ShapeDtypeStruct((N, C), jnp.float32),
                   jax.ShapeDtypeStruct((N, W128), jnp.float32)),
    )(na, we, wu0)


def _tc_edge(g, eaef, wr1, wr2, wperm):
    def body(g_ref, ee_ref, wr1_ref, wr2_ref, wp_ref, out_ref):
        ee = ee_ref[...]
        ef = ee[:, 4:12]
        hid = jnp.dot(ef, wr1_ref[...], preferred_element_type=jnp.float32)
        hid = hid * jax.nn.sigmoid(hid)
        rad = jnp.dot(hid, wr2_ref[...], preferred_element_type=jnp.float32)
        hm = g_ref[:, 0:C] * rad
        hm4 = jnp.concatenate(
            [hm, ee[:, 1:2] * hm, ee[:, 2:3] * hm, ee[:, 3:4] * hm], axis=1)
        msg = jnp.dot(hm4, wp_ref[...], preferred_element_type=jnp.float32)
        out_ref[...] = jnp.concatenate([msg, jnp.zeros_like(msg)], axis=1)

    return pl.pallas_call(
        body,
        grid=(E // BE_E,),
        in_specs=[pl.BlockSpec((BE_E, W128), lambda i: (i, 0)),
                  pl.BlockSpec((BE_E, 16), lambda i: (i, 0)),
                  pl.BlockSpec((8, 16), lambda i: (0, 0)),
                  pl.BlockSpec((16, C), lambda i: (0, 0)),
                  pl.BlockSpec((SH * C, C), lambda i: (0, 0))],
        out_specs=pl.BlockSpec((BE_E, W128), lambda i: (i, 0)),
        out_shape=jax.ShapeDtypeStruct((E, W128), jnp.float32),
    )(g, eaef, wr1, wr2, wperm)


def _tc_combine(nf, parts, wskip, wup_next):
    def body(nf_ref, p0_ref, p1_ref, ws_ref, wu_ref, nfn_ref, tab_ref):
        agg = p0_ref[0][:, 0:C] + p1_ref[0][:, 0:C]
        nfn = agg + jnp.dot(nf_ref[...], ws_ref[...], preferred_element_type=jnp.float32)
        nfn_ref[...] = nfn
        h = jnp.dot(nfn, wu_ref[...], preferred_element_type=jnp.float32)
        tab_ref[...] = jnp.concatenate([h, jnp.zeros_like(h)], axis=1)

    return pl.pallas_call(
        body,
        grid=(N // BN,),
        in_specs=[pl.BlockSpec((BN, C), lambda i: (i, 0)),
                  pl.BlockSpec((1, BN, W128), lambda i: (0, i, 0)),
                  pl.BlockSpec((1, BN, W128), lambda i: (1, i, 0)),
                  pl.BlockSpec((C, C), lambda i: (0, 0)),
                  pl.BlockSpec((C, C), lambda i: (0, 0))],
        out_specs=(pl.BlockSpec((BN, C), lambda i: (i, 0)),
                   pl.BlockSpec((BN, W128), lambda i: (i, 0))),
        out_shape=(jax.ShapeDtypeStruct((N, C), jnp.float32),
                   jax.ShapeDtypeStruct((N, W128), jnp.float32)),
    )(nf, parts, parts, wskip, wup_next)


def kernel(atom_pos, node_attrs, edge_index, shifts, W_embed, W_up, Wr1, Wr2, W_msg, W_skip):
    src = edge_index[0].astype(jnp.int32)
    dst = edge_index[1].astype(jnp.int32)
    pos128 = jnp.pad(atom_pos, ((0, 0), (0, W128 - 3)))
    sh8 = jnp.pad(shifts, ((0, 0), (0, 5)))
    zeros_nw = jnp.zeros((N, W128), jnp.float32)

    ps = _sc_gather(pos128, src)
    pd = _sc_gather(pos128, dst)
    eaef = _tc_geom(ps, pd, sh8)
    nf, tab = _tc_node_init(node_attrs, W_embed, W_up[0])

    nl = W_msg.shape[0]
    wperms = W_msg.reshape(nl, C, SH, C).transpose(0, 2, 1, 3).reshape(nl, SH * C, C)
    wup1 = W_up[1]

    def layer_step(carry, xs):
        nf_c, tab_c = carry
        wr1, wr2, wperm, wskip = xs
        g = _sc_gather(tab_c, src)
        msg = _tc_edge(g, eaef, wr1, wr2, wperm)
        parts = _sc_scatter(msg, dst, zeros_nw)
        nf_n, tab_n = _tc_combine(nf_c, parts, wskip, wup1)
        return (nf_n, tab_n), nf_n

    _, ys = lax.scan(layer_step, (nf, tab), (Wr1, Wr2, wperms, W_skip))
    return jnp.concatenate([ys[0], ys[1]], axis=-1)
